# baseline (speedup 1.0000x reference)
import functools

import jax
import jax.numpy as jnp
import numpy as np
from jax import lax
from jax.experimental import pallas as pl
from jax.experimental.pallas import tpu as pltpu
from jax.experimental.pallas import tpu_sc as plsc

N = 10000
K = 20
NSEG = 16
F = 16
NPAD = 10240
R = 256
C = 256
NB = NPAD // R
RE = 256
NBE = NPAD // RE
EB = K * RE
EPAD = K * NPAD
NE = float(N * K)
IMAX = np.int32(np.iinfo(np.int32).max)
_PREC = None


def _mm(a, b):
    return jnp.dot(a, b, precision=_PREC, preferred_element_type=jnp.float32)


KP = 24


def _knn_body(rowsT_ref, feat_hbm, rlo_ref, rhi_ref, lo_ref, nc_ref, out_ref,
              fv_ref, sem):
    rb = pl.program_id(0)

    @pl.when(rb == 0)
    def _():
        cp = pltpu.make_async_copy(feat_hbm, fv_ref, sem)
        cp.start()
        cp.wait()

    rowsT = rowsT_ref[...]
    sqr = jnp.sum(rowsT * rowsT, axis=0, keepdims=True)
    row_ids = rb * R + lax.broadcasted_iota(jnp.int32, (1, R), 1)
    rlo = rlo_ref[0]
    rhi = rhi_ref[0]
    lo = lo_ref[rb]
    nc = nc_ref[rb]

    init_val = jnp.full((KP, R), jnp.inf, jnp.float32)
    init_idx = jnp.full((KP, R), IMAX, jnp.int32)
    padv = jnp.full((KP - K, R), jnp.inf, jnp.float32)
    padi = jnp.full((KP - K, R), IMAX, jnp.int32)

    def chunk(j, carry):
        val, idxc = carry
        c0 = (lo + j) * C
        cols = fv_ref[pl.ds(c0, C), :]
        prod = lax.dot_general(cols, rowsT, (((1,), (0,)), ((), ())),
                               precision=_PREC,
                               preferred_element_type=jnp.float32)
        sqc = jnp.sum(cols * cols, axis=1, keepdims=True)
        d = sqc + sqr - 2.0 * prod
        cid = c0 + lax.broadcasted_iota(jnp.int32, (C, 1), 0)
        bad = (cid < rlo) | (cid >= rhi) | (cid == row_ids)
        d = jnp.where(bad, jnp.inf, d)
        cval = jnp.concatenate([val, d], axis=0)
        cidx = jnp.concatenate(
            [idxc, jnp.broadcast_to(cid, (C, R))], axis=0)
        nv, ni = [], []
        for _ in range(K):
            m = jnp.min(cval, axis=0, keepdims=True)
            sel = jnp.min(jnp.where(cval == m, cidx, IMAX),
                          axis=0, keepdims=True)
            nv.append(m)
            ni.append(sel)
            cval = jnp.where(cidx == sel, jnp.inf, cval)
        return (jnp.concatenate(nv + [padv], axis=0),
                jnp.concatenate(ni + [padi], axis=0))

    _, idx = lax.fori_loop(0, nc, chunk, (init_val, init_idx))
    out_ref[0] = jnp.clip(idx[:K, :], 0, N - 1)


def _knn(featT, feat, rlo, rhi, lo, nc):
    return pl.pallas_call(
        _knn_body,
        grid=(NB,),
        in_specs=[
            pl.BlockSpec((F, R), lambda rb: (0, rb)),
            pl.BlockSpec(memory_space=pltpu.MemorySpace.HBM),
            pl.BlockSpec((1, 1, R), lambda rb: (rb, 0, 0)),
            pl.BlockSpec((1, 1, R), lambda rb: (rb, 0, 0)),
            pl.BlockSpec(memory_space=pltpu.SMEM),
            pl.BlockSpec(memory_space=pltpu.SMEM),
        ],
        out_specs=pl.BlockSpec((1, K, R), lambda rb: (rb, 0, 0)),
        out_shape=jax.ShapeDtypeStruct((NB, K, R), jnp.int32),
        scratch_shapes=[
            pltpu.VMEM((NPAD, F), jnp.float32),
            pltpu.SemaphoreType.DMA,
        ],
    )(featT, feat, rlo, rhi, lo, nc)


def _gather_rows(table, idx_flat):
    info = plsc.get_sparse_core_info()
    nw = info.num_cores * info.num_subcores
    per_w = EPAD // nw
    ch = 128
    nch = per_w // ch
    mesh = plsc.VectorSubcoreMesh(core_axis_name="c", subcore_axis_name="s")

    @functools.partial(
        pl.kernel,
        out_type=jax.ShapeDtypeStruct((EPAD, F), jnp.float32),
        mesh=mesh,
        compiler_params=pltpu.CompilerParams(use_tc_tiling_on_sc=False),
        scratch_types=[
            pltpu.VMEM((nch, ch), jnp.int32),
            pltpu.VMEM((per_w, F), jnp.float32),
            pltpu.SemaphoreType.DMA,
            pltpu.SemaphoreType.DMA,
        ],
    )
    def gk(table_hbm, idx_hbm, out_hbm, idx_v, rows_v, semg, semb):
        wid = lax.axis_index("s") * info.num_cores + lax.axis_index("c")
        cp = pltpu.make_async_copy(idx_hbm.at[pl.ds(wid * nch, nch)],
                                   idx_v, semb)
        cp.start()
        cp.wait()

        def fire(j, carry):
            pltpu.async_copy(table_hbm.at[idx_v.at[j]],
                             rows_v.at[pl.ds(j * ch, ch)], semg)
            return carry

        lax.fori_loop(0, nch, fire, 0)

        def drain(j, carry):
            pltpu.make_async_copy(table_hbm.at[idx_v.at[j]],
                                  rows_v.at[pl.ds(j * ch, ch)], semg).wait()
            return carry

        lax.fori_loop(0, nch, drain, 0)
        out_cp = pltpu.make_async_copy(
            rows_v, out_hbm.at[pl.ds(wid * per_w, per_w)], semb)
        out_cp.start()
        out_cp.wait()

    return gk(table, idx_flat.reshape(EPAD // ch, ch))


def _eb_spec():
    return pl.BlockSpec((EB, 16), lambda nb: (nb, 0))


def _passA_body(pos_ref, g_ref, w_ref, b_ref, z_ref, s_ref, ss_ref):
    nb = pl.program_id(0)
    p = pos_ref[...]
    w = w_ref[...]
    b = b_ref[...]
    valid = (nb * RE + lax.broadcasted_iota(jnp.int32, (RE, 1), 0)) < N
    s_acc = jnp.zeros((1, 16), jnp.float32)
    ss_acc = jnp.zeros((1, 16), jnp.float32)
    for k in range(K):
        g = g_ref[k * RE:(k + 1) * RE, :]
        z = _mm(jnp.concatenate([p, g - p], axis=1), w) + b
        z_ref[k * RE:(k + 1) * RE, :] = z
        zm = jnp.where(valid, z, 0.0)
        s_acc += jnp.sum(zm, axis=0, keepdims=True)
        ss_acc += jnp.sum(zm * zm, axis=0, keepdims=True)

    @pl.when(nb == 0)
    def _():
        s_ref[...] = jnp.zeros_like(s_ref)
        ss_ref[...] = jnp.zeros_like(ss_ref)

    s_ref[...] += s_acc
    ss_ref[...] += ss_acc


def _passA(pos16, g1, w11p, b11):
    return pl.pallas_call(
        _passA_body,
        grid=(NBE,),
        in_specs=[
            pl.BlockSpec((RE, 16), lambda nb: (nb, 0)),
            _eb_spec(),
            pl.BlockSpec((32, 16), lambda nb: (0, 0)),
            pl.BlockSpec((1, 16), lambda nb: (0, 0)),
        ],
        out_specs=[
            _eb_spec(),
            pl.BlockSpec((1, 16), lambda nb: (0, 0)),
            pl.BlockSpec((1, 16), lambda nb: (0, 0)),
        ],
        out_shape=[
            jax.ShapeDtypeStruct((EPAD, 16), jnp.float32),
            jax.ShapeDtypeStruct((1, 16), jnp.float32),
            jax.ShapeDtypeStruct((1, 16), jnp.float32),
        ],
    )(pos16, g1, w11p, b11)


def _bn_scale_shift(s, ss, g, bb):
    mean = s / NE
    var = ss / NE - mean * mean
    scale = g / jnp.sqrt(var + 1e-5)
    return scale, bb - mean * scale


def _passB_body(z1_ref, s1_ref, ss1_ref, g11_ref, bb11_ref, w_ref, b_ref,
                z2_ref, s_ref, ss_ref):
    nb = pl.program_id(0)
    scale, shift = _bn_scale_shift(s1_ref[...], ss1_ref[...],
                                   g11_ref[...], bb11_ref[...])
    w = w_ref[...]
    b = b_ref[...]
    valid = (nb * RE + lax.broadcasted_iota(jnp.int32, (RE, 1), 0)) < N
    s_acc = jnp.zeros((1, 16), jnp.float32)
    ss_acc = jnp.zeros((1, 16), jnp.float32)
    for k in range(K):
        h = jnp.maximum(z1_ref[k * RE:(k + 1) * RE, :] * scale + shift, 0.0)
        z = _mm(h, w) + b
        z2_ref[k * RE:(k + 1) * RE, :] = z
        zm = jnp.where(valid, z, 0.0)
        s_acc += jnp.sum(zm, axis=0, keepdims=True)
        ss_acc += jnp.sum(zm * zm, axis=0, keepdims=True)

    @pl.when(nb == 0)
    def _():
        s_ref[...] = jnp.zeros_like(s_ref)
        ss_ref[...] = jnp.zeros_like(ss_ref)

    s_ref[...] += s_acc
    ss_ref[...] += ss_acc


def _passB(z1, s1, ss1, g11, bb11, w12, b12):
    vec = pl.BlockSpec((1, 16), lambda nb: (0, 0))
    return pl.pallas_call(
        _passB_body,
        grid=(NBE,),
        in_specs=[_eb_spec(), vec, vec, vec, vec,
                  pl.BlockSpec((16, 16), lambda nb: (0, 0)), vec],
        out_specs=[_eb_spec(), vec, vec],
        out_shape=[
            jax.ShapeDtypeStruct((EPAD, 16), jnp.float32),
            jax.ShapeDtypeStruct((1, 16), jnp.float32),
            jax.ShapeDtypeStruct((1, 16), jnp.float32),
        ],
    )(z1, s1, ss1, g11, bb11, w12, b12)


def _passC_body(z2_ref, s2_ref, ss2_ref, g12_ref, bb12_ref, w_ref, b_ref,
                x1_ref):
    scale, shift = _bn_scale_shift(s2_ref[...], ss2_ref[...],
                                   g12_ref[...], bb12_ref[...])
    w = w_ref[...]
    b = b_ref[...]
    acc = None
    for k in range(K):
        h = jnp.maximum(z2_ref[k * RE:(k + 1) * RE, :] * scale + shift, 0.0)
        m = _mm(h, w) + b
        acc = m if acc is None else jnp.maximum(acc, m)
    x1_ref[...] = acc


def _passC(z2, s2, ss2, g12, bb12, w13, b13):
    vec = pl.BlockSpec((1, 16), lambda nb: (0, 0))
    return pl.pallas_call(
        _passC_body,
        grid=(NBE,),
        in_specs=[_eb_spec(), vec, vec, vec, vec,
                  pl.BlockSpec((16, 16), lambda nb: (0, 0)), vec],
        out_specs=pl.BlockSpec((RE, 16), lambda nb: (nb, 0)),
        out_shape=jax.ShapeDtypeStruct((NPAD, 16), jnp.float32),
    )(z2, s2, ss2, g12, bb12, w13, b13)


def _passD_body(x1_ref, g_ref, w_ref, b_ref, x2_ref):
    x1b = x1_ref[...]
    w = w_ref[...]
    b = b_ref[...]
    acc = None
    for k in range(K):
        g = g_ref[k * RE:(k + 1) * RE, :]
        m = _mm(jnp.concatenate([x1b, g - x1b], axis=1), w) + b
        acc = m if acc is None else jnp.maximum(acc, m)
    x2_ref[...] = acc


def _passD(x1, g2, w21, b21):
    return pl.pallas_call(
        _passD_body,
        grid=(NBE,),
        in_specs=[
            pl.BlockSpec((RE, 16), lambda nb: (nb, 0)),
            _eb_spec(),
            pl.BlockSpec((32, 32), lambda nb: (0, 0)),
            pl.BlockSpec((1, 32), lambda nb: (0, 0)),
        ],
        out_specs=pl.BlockSpec((RE, 32), lambda nb: (nb, 0)),
        out_shape=jax.ShapeDtypeStruct((NPAD, 32), jnp.float32),
    )(x1, g2, w21, b21)


def _pool_body(x1_ref, x2_ref, bcol_ref, w_ref, b_ref, pool_ref):
    nb = pl.program_id(0)
    o = _mm(jnp.concatenate([x1_ref[...], x2_ref[...]], axis=1),
            w_ref[...]) + b_ref[...]
    bcol = bcol_ref[...]
    parts = [
        jnp.max(jnp.where(bcol == s, o, -jnp.inf), axis=0, keepdims=True)
        for s in range(NSEG)
    ]
    pooled = jnp.concatenate(parts, axis=0)

    @pl.when(nb == 0)
    def _():
        pool_ref[...] = pooled

    @pl.when(nb > 0)
    def _():
        pool_ref[...] = jnp.maximum(pool_ref[...], pooled)


def _pool(x1, x2, bcol, wl, bl):
    return pl.pallas_call(
        _pool_body,
        grid=(NBE,),
        in_specs=[
            pl.BlockSpec((RE, 16), lambda nb: (nb, 0)),
            pl.BlockSpec((RE, 32), lambda nb: (nb, 0)),
            pl.BlockSpec((RE, 1), lambda nb: (nb, 0)),
            pl.BlockSpec((48, 128), lambda nb: (0, 0)),
            pl.BlockSpec((1, 128), lambda nb: (0, 0)),
        ],
        out_specs=pl.BlockSpec((NSEG, 128), lambda nb: (0, 0)),
        out_shape=jax.ShapeDtypeStruct((NSEG, 128), jnp.float32),
    )(x1, x2, bcol, wl, bl)


def _head_body(p_ref, w31_ref, b31_ref, w32_ref, b32_ref, w33_ref, b33_ref,
               out_ref):
    h = jnp.maximum(_mm(p_ref[...], w31_ref[...]) + b31_ref[...], 0.0)
    h = jnp.maximum(_mm(h, w32_ref[...]) + b32_ref[...], 0.0)
    z = _mm(h, w33_ref[...]) + b33_ref[...]
    mx = jnp.max(z, axis=1, keepdims=True)
    e = z - mx
    out_ref[...] = e - jnp.log(jnp.sum(jnp.exp(e), axis=1, keepdims=True))


def _head(pooled, w31, b31, w32, b32, w33, b33):
    return pl.pallas_call(
        _head_body,
        out_shape=jax.ShapeDtypeStruct((NSEG, 40), jnp.float32),
    )(pooled, w31, b31, w32, b32, w33, b33)


def _edge_order(idx):
    return idx.reshape(-1)


def kernel(pos, batch, w11, b11, g11, bb11, w12, b12, g12, bb12, w13, b13,
           w21, b21, wl, bl, w31, b31, w32, b32, w33, b33):
    batch = batch.astype(jnp.int32)

    pos16 = jnp.zeros((NPAD, 16), jnp.float32).at[:N, :3].set(pos)
    pos16T = pos16.T
    bcol = jnp.full((NPAD, 1), NSEG, jnp.int32).at[:N, 0].set(batch)

    ar = jnp.arange(NSEG)
    seg_start = jnp.sum(batch[None, :] < ar[:, None], axis=1).astype(jnp.int32)
    seg_end = jnp.sum(batch[None, :] <= ar[:, None], axis=1).astype(jnp.int32)
    rlo = jnp.zeros((NPAD,), jnp.int32).at[:N].set(
        seg_start[batch]).reshape(NB, 1, R)
    rhi = jnp.zeros((NPAD,), jnp.int32).at[:N].set(
        seg_end[batch]).reshape(NB, 1, R)
    blk0 = jnp.arange(NB, dtype=jnp.int32) * R
    bvec = bcol[:, 0]
    b_lo = bvec[blk0]
    b_hi = bvec[jnp.minimum(blk0 + R - 1, NPAD - 1)]
    lo_col = seg_start[jnp.minimum(b_lo, NSEG - 1)]
    hi_col = seg_end[jnp.minimum(b_hi, NSEG - 1)]
    lo_blk = lo_col // C
    nc = jnp.maximum((hi_col - lo_blk * C + C - 1) // C, 0)
    nc = jnp.where(b_lo >= NSEG, 0, nc).astype(jnp.int32)
    lo_blk = lo_blk.astype(jnp.int32)

    w11p = jnp.zeros((32, 16), jnp.float32)
    w11p = w11p.at[0:3].set(w11[0:3]).at[16:19].set(w11[3:6])

    r2 = lambda v: v.reshape(1, -1)

    idx1 = _knn(pos16T, pos16, rlo, rhi, lo_blk, nc)
    g1 = _gather_rows(pos16, _edge_order(idx1))
    z1, s1, ss1 = _passA(pos16, g1, w11p, r2(b11))
    z2, s2, ss2 = _passB(z1, s1, ss1, r2(g11), r2(bb11), w12, r2(b12))
    x1 = _passC(z2, s2, ss2, r2(g12), r2(bb12), w13, r2(b13))

    idx2 = _knn(x1.T, x1, rlo, rhi, lo_blk, nc)
    g2 = _gather_rows(x1, _edge_order(idx2))
    x2 = _passD(x1, g2, w21, r2(b21))

    pooled = _pool(x1, x2, bcol, wl, r2(bl))
    return _head(pooled, w31, r2(b31), w32, r2(b32), w33, r2(b33))


# speedup vs baseline: 1.0508x; 1.0508x over previous
import functools

import jax
import jax.numpy as jnp
import numpy as np
from jax import lax
from jax.experimental import pallas as pl
from jax.experimental.pallas import tpu as pltpu
from jax.experimental.pallas import tpu_sc as plsc

N = 10000
K = 20
NSEG = 16
F = 16
NPAD = 10240
R = 256
C = 256
NB = NPAD // R
RE = 1024
NBE = NPAD // RE
EB = K * RE
EPAD = K * NPAD
NE = float(N * K)
IMAX = np.int32(np.iinfo(np.int32).max)
_PREC = None


def _mm(a, b):
    return jnp.dot(a, b, precision=_PREC, preferred_element_type=jnp.float32)


KP = 24


def _knn_body(rowsT_ref, feat_hbm, rlo_ref, rhi_ref, lo_ref, nc_ref, out_ref,
              fv_ref, sem):
    rb = pl.program_id(0)

    @pl.when(rb == 0)
    def _():
        cp = pltpu.make_async_copy(feat_hbm, fv_ref, sem)
        cp.start()
        cp.wait()

    rowsT = rowsT_ref[...]
    sqr = jnp.sum(rowsT * rowsT, axis=0, keepdims=True)
    row_ids = rb * R + lax.broadcasted_iota(jnp.int32, (1, R), 1)
    rlo = rlo_ref[0]
    rhi = rhi_ref[0]
    lo = lo_ref[rb]
    nc = nc_ref[rb]

    init_val = jnp.full((KP, R), jnp.inf, jnp.float32)
    init_idx = jnp.full((KP, R), IMAX, jnp.int32)
    padv = jnp.full((KP - K, R), jnp.inf, jnp.float32)
    padi = jnp.full((KP - K, R), IMAX, jnp.int32)

    def chunk(j, carry):
        val, idxc = carry
        c0 = (lo + j) * C
        cols = fv_ref[pl.ds(c0, C), :]
        prod = lax.dot_general(cols, rowsT, (((1,), (0,)), ((), ())),
                               precision=_PREC,
                               preferred_element_type=jnp.float32)
        sqc = jnp.sum(cols * cols, axis=1, keepdims=True)
        d = sqc + sqr - 2.0 * prod
        cid = c0 + lax.broadcasted_iota(jnp.int32, (C, 1), 0)
        bad = (cid < rlo) | (cid >= rhi) | (cid == row_ids)
        d = jnp.where(bad, jnp.inf, d)
        cval = jnp.concatenate([val, d], axis=0)
        cidx = jnp.concatenate(
            [idxc, jnp.broadcast_to(cid, (C, R))], axis=0)
        nv, ni = [], []
        for _ in range(K):
            m = jnp.min(cval, axis=0, keepdims=True)
            sel = jnp.min(jnp.where(cval == m, cidx, IMAX),
                          axis=0, keepdims=True)
            nv.append(m)
            ni.append(sel)
            cval = jnp.where(cidx == sel, jnp.inf, cval)
        return (jnp.concatenate(nv + [padv], axis=0),
                jnp.concatenate(ni + [padi], axis=0))

    _, idx = lax.fori_loop(0, nc, chunk, (init_val, init_idx))
    out_ref[0] = jnp.clip(idx[:K, :], 0, N - 1)


def _knn(featT, feat, rlo, rhi, lo, nc):
    return pl.pallas_call(
        _knn_body,
        grid=(NB,),
        in_specs=[
            pl.BlockSpec((F, R), lambda rb: (0, rb)),
            pl.BlockSpec(memory_space=pltpu.MemorySpace.HBM),
            pl.BlockSpec((1, 1, R), lambda rb: (rb, 0, 0)),
            pl.BlockSpec((1, 1, R), lambda rb: (rb, 0, 0)),
            pl.BlockSpec(memory_space=pltpu.SMEM),
            pl.BlockSpec(memory_space=pltpu.SMEM),
        ],
        out_specs=pl.BlockSpec((1, K, R), lambda rb: (rb, 0, 0)),
        out_shape=jax.ShapeDtypeStruct((NB, K, R), jnp.int32),
        scratch_shapes=[
            pltpu.VMEM((NPAD, F), jnp.float32),
            pltpu.SemaphoreType.DMA,
        ],
    )(featT, feat, rlo, rhi, lo, nc)


def _gather_rows(table, idx_flat):
    info = plsc.get_sparse_core_info()
    nw = info.num_cores * info.num_subcores
    per_w = EPAD // nw
    ch = 128
    nch = per_w // ch
    mesh = plsc.VectorSubcoreMesh(core_axis_name="c", subcore_axis_name="s")

    @functools.partial(
        pl.kernel,
        out_type=jax.ShapeDtypeStruct((EPAD, F), jnp.float32),
        mesh=mesh,
        compiler_params=pltpu.CompilerParams(use_tc_tiling_on_sc=False),
        scratch_types=[
            pltpu.VMEM((nch, ch), jnp.int32),
            pltpu.VMEM((per_w, F), jnp.float32),
            pltpu.SemaphoreType.DMA,
            pltpu.SemaphoreType.DMA,
        ],
    )
    def gk(table_hbm, idx_hbm, out_hbm, idx_v, rows_v, semg, semb):
        wid = lax.axis_index("s") * info.num_cores + lax.axis_index("c")
        cp = pltpu.make_async_copy(idx_hbm.at[pl.ds(wid * nch, nch)],
                                   idx_v, semb)
        cp.start()
        cp.wait()

        def fire(j, carry):
            pltpu.async_copy(table_hbm.at[idx_v.at[j]],
                             rows_v.at[pl.ds(j * ch, ch)], semg)
            return carry

        lax.fori_loop(0, nch, fire, 0)

        def drain(j, carry):
            pltpu.make_async_copy(table_hbm.at[idx_v.at[j]],
                                  rows_v.at[pl.ds(j * ch, ch)], semg).wait()
            return carry

        lax.fori_loop(0, nch, drain, 0)
        out_cp = pltpu.make_async_copy(
            rows_v, out_hbm.at[pl.ds(wid * per_w, per_w)], semb)
        out_cp.start()
        out_cp.wait()

    return gk(table, idx_flat.reshape(EPAD // ch, ch))


def _eb_spec():
    return pl.BlockSpec((EB, 16), lambda nb: (nb, 0))


def _passA_body(pos_ref, g_ref, w_ref, b_ref, z_ref, s_ref, ss_ref):
    nb = pl.program_id(0)
    w = w_ref[...]
    b = b_ref[...]
    s_acc = jnp.zeros((1, 16), jnp.float32)
    ss_acc = jnp.zeros((1, 16), jnp.float32)
    for q in range(RE // R):
        p = pos_ref[q * R:(q + 1) * R, :]
        valid = (nb * RE + q * R
                 + lax.broadcasted_iota(jnp.int32, (R, 1), 0)) < N
        for k in range(K):
            s = q * K + k
            g = g_ref[s * R:(s + 1) * R, :]
            z = _mm(jnp.concatenate([p, g - p], axis=1), w) + b
            z_ref[s * R:(s + 1) * R, :] = z
            zm = jnp.where(valid, z, 0.0)
            s_acc += jnp.sum(zm, axis=0, keepdims=True)
            ss_acc += jnp.sum(zm * zm, axis=0, keepdims=True)

    @pl.when(nb == 0)
    def _():
        s_ref[...] = jnp.zeros_like(s_ref)
        ss_ref[...] = jnp.zeros_like(ss_ref)

    s_ref[...] += s_acc
    ss_ref[...] += ss_acc


def _passA(pos16, g1, w11p, b11):
    return pl.pallas_call(
        _passA_body,
        grid=(NBE,),
        in_specs=[
            pl.BlockSpec((RE, 16), lambda nb: (nb, 0)),
            _eb_spec(),
            pl.BlockSpec((32, 16), lambda nb: (0, 0)),
            pl.BlockSpec((1, 16), lambda nb: (0, 0)),
        ],
        out_specs=[
            _eb_spec(),
            pl.BlockSpec((1, 16), lambda nb: (0, 0)),
            pl.BlockSpec((1, 16), lambda nb: (0, 0)),
        ],
        out_shape=[
            jax.ShapeDtypeStruct((EPAD, 16), jnp.float32),
            jax.ShapeDtypeStruct((1, 16), jnp.float32),
            jax.ShapeDtypeStruct((1, 16), jnp.float32),
        ],
    )(pos16, g1, w11p, b11)


def _bn_scale_shift(s, ss, g, bb):
    mean = s / NE
    var = ss / NE - mean * mean
    scale = g / jnp.sqrt(var + 1e-5)
    return scale, bb - mean * scale


def _passB_body(z1_ref, s1_ref, ss1_ref, g11_ref, bb11_ref, w_ref, b_ref,
                z2_ref, s_ref, ss_ref):
    nb = pl.program_id(0)
    scale, shift = _bn_scale_shift(s1_ref[...], ss1_ref[...],
                                   g11_ref[...], bb11_ref[...])
    w = w_ref[...]
    b = b_ref[...]
    s_acc = jnp.zeros((1, 16), jnp.float32)
    ss_acc = jnp.zeros((1, 16), jnp.float32)
    for q in range(RE // R):
        valid = (nb * RE + q * R
                 + lax.broadcasted_iota(jnp.int32, (R, 1), 0)) < N
        for k in range(K):
            s = q * K + k
            h = jnp.maximum(z1_ref[s * R:(s + 1) * R, :] * scale + shift,
                            0.0)
            z = _mm(h, w) + b
            z2_ref[s * R:(s + 1) * R, :] = z
            zm = jnp.where(valid, z, 0.0)
            s_acc += jnp.sum(zm, axis=0, keepdims=True)
            ss_acc += jnp.sum(zm * zm, axis=0, keepdims=True)

    @pl.when(nb == 0)
    def _():
        s_ref[...] = jnp.zeros_like(s_ref)
        ss_ref[...] = jnp.zeros_like(ss_ref)

    s_ref[...] += s_acc
    ss_ref[...] += ss_acc


def _passB(z1, s1, ss1, g11, bb11, w12, b12):
    vec = pl.BlockSpec((1, 16), lambda nb: (0, 0))
    return pl.pallas_call(
        _passB_body,
        grid=(NBE,),
        in_specs=[_eb_spec(), vec, vec, vec, vec,
                  pl.BlockSpec((16, 16), lambda nb: (0, 0)), vec],
        out_specs=[_eb_spec(), vec, vec],
        out_shape=[
            jax.ShapeDtypeStruct((EPAD, 16), jnp.float32),
            jax.ShapeDtypeStruct((1, 16), jnp.float32),
            jax.ShapeDtypeStruct((1, 16), jnp.float32),
        ],
    )(z1, s1, ss1, g11, bb11, w12, b12)


def _passC_body(z2_ref, s2_ref, ss2_ref, g12_ref, bb12_ref, w_ref, b_ref,
                x1_ref):
    scale, shift = _bn_scale_shift(s2_ref[...], ss2_ref[...],
                                   g12_ref[...], bb12_ref[...])
    w = w_ref[...]
    b = b_ref[...]
    for q in range(RE // R):
        acc = None
        for k in range(K):
            s = q * K + k
            h = jnp.maximum(z2_ref[s * R:(s + 1) * R, :] * scale + shift,
                            0.0)
            m = _mm(h, w) + b
            acc = m if acc is None else jnp.maximum(acc, m)
        x1_ref[q * R:(q + 1) * R, :] = acc


def _passC(z2, s2, ss2, g12, bb12, w13, b13):
    vec = pl.BlockSpec((1, 16), lambda nb: (0, 0))
    return pl.pallas_call(
        _passC_body,
        grid=(NBE,),
        in_specs=[_eb_spec(), vec, vec, vec, vec,
                  pl.BlockSpec((16, 16), lambda nb: (0, 0)), vec],
        out_specs=pl.BlockSpec((RE, 16), lambda nb: (nb, 0)),
        out_shape=jax.ShapeDtypeStruct((NPAD, 16), jnp.float32),
    )(z2, s2, ss2, g12, bb12, w13, b13)


def _passD_body(x1_ref, g_ref, w_ref, b_ref, x2_ref):
    w = w_ref[...]
    b = b_ref[...]
    for q in range(RE // R):
        x1b = x1_ref[q * R:(q + 1) * R, :]
        acc = None
        for k in range(K):
            s = q * K + k
            g = g_ref[s * R:(s + 1) * R, :]
            m = _mm(jnp.concatenate([x1b, g - x1b], axis=1), w) + b
            acc = m if acc is None else jnp.maximum(acc, m)
        x2_ref[q * R:(q + 1) * R, :] = acc


def _passD(x1, g2, w21, b21):
    return pl.pallas_call(
        _passD_body,
        grid=(NBE,),
        in_specs=[
            pl.BlockSpec((RE, 16), lambda nb: (nb, 0)),
            _eb_spec(),
            pl.BlockSpec((32, 32), lambda nb: (0, 0)),
            pl.BlockSpec((1, 32), lambda nb: (0, 0)),
        ],
        out_specs=pl.BlockSpec((RE, 32), lambda nb: (nb, 0)),
        out_shape=jax.ShapeDtypeStruct((NPAD, 32), jnp.float32),
    )(x1, g2, w21, b21)


def _pool_body(x1_ref, x2_ref, bcol_ref, w_ref, b_ref, pool_ref):
    nb = pl.program_id(0)
    o = _mm(jnp.concatenate([x1_ref[...], x2_ref[...]], axis=1),
            w_ref[...]) + b_ref[...]
    bcol = bcol_ref[...]
    parts = [
        jnp.max(jnp.where(bcol == s, o, -jnp.inf), axis=0, keepdims=True)
        for s in range(NSEG)
    ]
    pooled = jnp.concatenate(parts, axis=0)

    @pl.when(nb == 0)
    def _():
        pool_ref[...] = pooled

    @pl.when(nb > 0)
    def _():
        pool_ref[...] = jnp.maximum(pool_ref[...], pooled)


def _pool(x1, x2, bcol, wl, bl):
    return pl.pallas_call(
        _pool_body,
        grid=(NBE,),
        in_specs=[
            pl.BlockSpec((RE, 16), lambda nb: (nb, 0)),
            pl.BlockSpec((RE, 32), lambda nb: (nb, 0)),
            pl.BlockSpec((RE, 1), lambda nb: (nb, 0)),
            pl.BlockSpec((48, 128), lambda nb: (0, 0)),
            pl.BlockSpec((1, 128), lambda nb: (0, 0)),
        ],
        out_specs=pl.BlockSpec((NSEG, 128), lambda nb: (0, 0)),
        out_shape=jax.ShapeDtypeStruct((NSEG, 128), jnp.float32),
    )(x1, x2, bcol, wl, bl)


def _head_body(p_ref, w31_ref, b31_ref, w32_ref, b32_ref, w33_ref, b33_ref,
               out_ref):
    h = jnp.maximum(_mm(p_ref[...], w31_ref[...]) + b31_ref[...], 0.0)
    h = jnp.maximum(_mm(h, w32_ref[...]) + b32_ref[...], 0.0)
    z = _mm(h, w33_ref[...]) + b33_ref[...]
    mx = jnp.max(z, axis=1, keepdims=True)
    e = z - mx
    out_ref[...] = e - jnp.log(jnp.sum(jnp.exp(e), axis=1, keepdims=True))


def _head(pooled, w31, b31, w32, b32, w33, b33):
    return pl.pallas_call(
        _head_body,
        out_shape=jax.ShapeDtypeStruct((NSEG, 40), jnp.float32),
    )(pooled, w31, b31, w32, b32, w33, b33)


def _edge_order(idx):
    return idx.reshape(-1)


def kernel(pos, batch, w11, b11, g11, bb11, w12, b12, g12, bb12, w13, b13,
           w21, b21, wl, bl, w31, b31, w32, b32, w33, b33):
    batch = batch.astype(jnp.int32)

    pos16 = jnp.zeros((NPAD, 16), jnp.float32).at[:N, :3].set(pos)
    pos16T = pos16.T
    bcol = jnp.full((NPAD, 1), NSEG, jnp.int32).at[:N, 0].set(batch)

    ar = jnp.arange(NSEG)
    seg_start = jnp.sum(batch[None, :] < ar[:, None], axis=1).astype(jnp.int32)
    seg_end = jnp.sum(batch[None, :] <= ar[:, None], axis=1).astype(jnp.int32)
    rlo = jnp.zeros((NPAD,), jnp.int32).at[:N].set(
        seg_start[batch]).reshape(NB, 1, R)
    rhi = jnp.zeros((NPAD,), jnp.int32).at[:N].set(
        seg_end[batch]).reshape(NB, 1, R)
    blk0 = jnp.arange(NB, dtype=jnp.int32) * R
    bvec = bcol[:, 0]
    b_lo = bvec[blk0]
    b_hi = bvec[jnp.minimum(blk0 + R - 1, NPAD - 1)]
    lo_col = seg_start[jnp.minimum(b_lo, NSEG - 1)]
    hi_col = seg_end[jnp.minimum(b_hi, NSEG - 1)]
    lo_blk = lo_col // C
    nc = jnp.maximum((hi_col - lo_blk * C + C - 1) // C, 0)
    nc = jnp.where(b_lo >= NSEG, 0, nc).astype(jnp.int32)
    lo_blk = lo_blk.astype(jnp.int32)

    w11p = jnp.zeros((32, 16), jnp.float32)
    w11p = w11p.at[0:3].set(w11[0:3]).at[16:19].set(w11[3:6])

    r2 = lambda v: v.reshape(1, -1)

    idx1 = _knn(pos16T, pos16, rlo, rhi, lo_blk, nc)
    g1 = _gather_rows(pos16, _edge_order(idx1))
    z1, s1, ss1 = _passA(pos16, g1, w11p, r2(b11))
    z2, s2, ss2 = _passB(z1, s1, ss1, r2(g11), r2(bb11), w12, r2(b12))
    x1 = _passC(z2, s2, ss2, r2(g12), r2(bb12), w13, r2(b13))

    idx2 = _knn(x1.T, x1, rlo, rhi, lo_blk, nc)
    g2 = _gather_rows(x1, _edge_order(idx2))
    x2 = _passD(x1, g2, w21, r2(b21))

    pooled = _pool(x1, x2, bcol, wl, r2(bl))
    return _head(pooled, w31, r2(b31), w32, r2(b32), w33, r2(b33))


# speedup vs baseline: 1.0660x; 1.0145x over previous
import functools

import jax
import jax.numpy as jnp
import numpy as np
from jax import lax
from jax.experimental import pallas as pl
from jax.experimental.pallas import tpu as pltpu
from jax.experimental.pallas import tpu_sc as plsc

N = 10000
K = 20
NSEG = 16
F = 16
NPAD = 10240
R = 256
C = 256
NB = NPAD // R
RE = 1024
NBE = NPAD // RE
EB = K * RE
EPAD = K * NPAD
NE = float(N * K)
IMAX = np.int32(np.iinfo(np.int32).max)
_PREC = None


def _mm(a, b):
    return jnp.dot(a, b, precision=_PREC, preferred_element_type=jnp.float32)


KP = 24


def _knn_body(rowsT_ref, feat_hbm, rlo_ref, rhi_ref, lo_ref, nc_ref, out_ref,
              fv_ref, sem):
    rb = pl.program_id(0)

    @pl.when(rb == 0)
    def _():
        cp = pltpu.make_async_copy(feat_hbm, fv_ref, sem)
        cp.start()
        cp.wait()

    rowsT = rowsT_ref[...]
    sqr = jnp.sum(rowsT * rowsT, axis=0, keepdims=True)
    row_ids = rb * R + lax.broadcasted_iota(jnp.int32, (1, R), 1)
    rlo = rlo_ref[0]
    rhi = rhi_ref[0]
    lo = lo_ref[rb]
    nc = nc_ref[rb]

    init_val = jnp.full((KP, R), jnp.inf, jnp.float32)
    init_idx = jnp.full((KP, R), IMAX, jnp.int32)
    padv = jnp.full((KP - K, R), jnp.inf, jnp.float32)
    padi = jnp.full((KP - K, R), IMAX, jnp.int32)

    def chunk(j, carry):
        val, idxc = carry
        c0 = (lo + j) * C
        cols = fv_ref[pl.ds(c0, C), :]
        prod = lax.dot_general(cols, rowsT, (((1,), (0,)), ((), ())),
                               precision=_PREC,
                               preferred_element_type=jnp.float32)
        sqc = jnp.sum(cols * cols, axis=1, keepdims=True)
        d = sqc + sqr - 2.0 * prod
        cid = c0 + lax.broadcasted_iota(jnp.int32, (C, 1), 0)
        bad = (cid < rlo) | (cid >= rhi) | (cid == row_ids)
        d = jnp.where(bad, jnp.inf, d)
        cval = jnp.concatenate([val, d], axis=0)
        cidx = jnp.concatenate(
            [idxc, jnp.broadcast_to(cid, (C, R))], axis=0)
        nv, ni = [], []
        for _ in range(K):
            m = jnp.min(cval, axis=0, keepdims=True)
            sel = jnp.min(jnp.where(cval == m, cidx, IMAX),
                          axis=0, keepdims=True)
            nv.append(m)
            ni.append(sel)
            cval = jnp.where(cidx == sel, jnp.inf, cval)
        return (jnp.concatenate(nv + [padv], axis=0),
                jnp.concatenate(ni + [padi], axis=0))

    _, idx = lax.fori_loop(0, nc, chunk, (init_val, init_idx))
    out_ref[0] = jnp.clip(idx[:K, :], 0, N - 1)


def _knn(featT, feat, rlo, rhi, lo, nc):
    return pl.pallas_call(
        _knn_body,
        grid=(NB,),
        in_specs=[
            pl.BlockSpec((F, R), lambda rb: (0, rb)),
            pl.BlockSpec(memory_space=pltpu.MemorySpace.HBM),
            pl.BlockSpec((1, 1, R), lambda rb: (rb, 0, 0)),
            pl.BlockSpec((1, 1, R), lambda rb: (rb, 0, 0)),
            pl.BlockSpec(memory_space=pltpu.SMEM),
            pl.BlockSpec(memory_space=pltpu.SMEM),
        ],
        out_specs=pl.BlockSpec((1, K, R), lambda rb: (rb, 0, 0)),
        out_shape=jax.ShapeDtypeStruct((NB, K, R), jnp.int32),
        scratch_shapes=[
            pltpu.VMEM((NPAD, F), jnp.float32),
            pltpu.SemaphoreType.DMA,
        ],
    )(featT, feat, rlo, rhi, lo, nc)


def _gather_rows(table, idx_flat):
    info = plsc.get_sparse_core_info()
    nw = info.num_cores * info.num_subcores
    per_w = EPAD // nw
    ch = 128
    nch = per_w // ch
    mesh = plsc.VectorSubcoreMesh(core_axis_name="c", subcore_axis_name="s")

    @functools.partial(
        pl.kernel,
        out_type=jax.ShapeDtypeStruct((EPAD, F), jnp.float32),
        mesh=mesh,
        compiler_params=pltpu.CompilerParams(use_tc_tiling_on_sc=False),
        scratch_types=[
            pltpu.VMEM((nch, ch), jnp.int32),
            pltpu.VMEM((per_w, F), jnp.float32),
            pltpu.SemaphoreType.DMA,
            pltpu.SemaphoreType.DMA,
        ],
    )
    def gk(table_hbm, idx_hbm, out_hbm, idx_v, rows_v, semg, semb):
        wid = lax.axis_index("s") * info.num_cores + lax.axis_index("c")
        cp = pltpu.make_async_copy(idx_hbm.at[pl.ds(wid * nch, nch)],
                                   idx_v, semb)
        cp.start()
        cp.wait()

        def fire(j, carry):
            pltpu.async_copy(table_hbm.at[idx_v.at[j]],
                             rows_v.at[pl.ds(j * ch, ch)], semg)
            return carry

        lax.fori_loop(0, nch, fire, 0)

        def drain(j, carry):
            pltpu.make_async_copy(table_hbm.at[idx_v.at[j]],
                                  rows_v.at[pl.ds(j * ch, ch)], semg).wait()
            return carry

        lax.fori_loop(0, nch, drain, 0)
        out_cp = pltpu.make_async_copy(
            rows_v, out_hbm.at[pl.ds(wid * per_w, per_w)], semb)
        out_cp.start()
        out_cp.wait()

    return gk(table, idx_flat.reshape(EPAD // ch, ch))


def _eb_spec():
    return pl.BlockSpec((EB, 16), lambda nb: (nb, 0))


def _passA_body(pos_ref, g_ref, w_ref, b_ref, z_ref, s_ref, ss_ref):
    nb = pl.program_id(0)
    w = w_ref[...]
    b = b_ref[...]
    s_acc = jnp.zeros((1, 16), jnp.float32)
    ss_acc = jnp.zeros((1, 16), jnp.float32)
    for q in range(RE // R):
        p = pos_ref[q * R:(q + 1) * R, :]
        valid = (nb * RE + q * R
                 + lax.broadcasted_iota(jnp.int32, (R, 1), 0)) < N
        for k in range(K):
            s = q * K + k
            g = g_ref[s * R:(s + 1) * R, :]
            z = _mm(jnp.concatenate([p, g - p], axis=1), w) + b
            z_ref[s * R:(s + 1) * R, :] = z
            zm = jnp.where(valid, z, 0.0)
            s_acc += jnp.sum(zm, axis=0, keepdims=True)
            ss_acc += jnp.sum(zm * zm, axis=0, keepdims=True)

    @pl.when(nb == 0)
    def _():
        s_ref[...] = jnp.zeros_like(s_ref)
        ss_ref[...] = jnp.zeros_like(ss_ref)

    s_ref[...] += s_acc
    ss_ref[...] += ss_acc


def _passA(pos16, g1, w11p, b11):
    return pl.pallas_call(
        _passA_body,
        grid=(NBE,),
        in_specs=[
            pl.BlockSpec((RE, 16), lambda nb: (nb, 0)),
            _eb_spec(),
            pl.BlockSpec((32, 16), lambda nb: (0, 0)),
            pl.BlockSpec((1, 16), lambda nb: (0, 0)),
        ],
        out_specs=[
            _eb_spec(),
            pl.BlockSpec((1, 16), lambda nb: (0, 0)),
            pl.BlockSpec((1, 16), lambda nb: (0, 0)),
        ],
        out_shape=[
            jax.ShapeDtypeStruct((EPAD, 16), jnp.float32),
            jax.ShapeDtypeStruct((1, 16), jnp.float32),
            jax.ShapeDtypeStruct((1, 16), jnp.float32),
        ],
    )(pos16, g1, w11p, b11)


def _bn_scale_shift(s, ss, g, bb):
    mean = s / NE
    var = ss / NE - mean * mean
    scale = g / jnp.sqrt(var + 1e-5)
    return scale, bb - mean * scale


def _passB_body(z1_ref, s1_ref, ss1_ref, g11_ref, bb11_ref, w_ref, b_ref,
                z2_ref, s_ref, ss_ref):
    nb = pl.program_id(0)
    scale, shift = _bn_scale_shift(s1_ref[...], ss1_ref[...],
                                   g11_ref[...], bb11_ref[...])
    w = w_ref[...]
    b = b_ref[...]
    s_acc = jnp.zeros((1, 16), jnp.float32)
    ss_acc = jnp.zeros((1, 16), jnp.float32)
    for q in range(RE // R):
        valid = (nb * RE + q * R
                 + lax.broadcasted_iota(jnp.int32, (R, 1), 0)) < N
        for k in range(K):
            s = q * K + k
            h = jnp.maximum(z1_ref[s * R:(s + 1) * R, :] * scale + shift,
                            0.0)
            z = _mm(h, w) + b
            z2_ref[s * R:(s + 1) * R, :] = z
            zm = jnp.where(valid, z, 0.0)
            s_acc += jnp.sum(zm, axis=0, keepdims=True)
            ss_acc += jnp.sum(zm * zm, axis=0, keepdims=True)

    @pl.when(nb == 0)
    def _():
        s_ref[...] = jnp.zeros_like(s_ref)
        ss_ref[...] = jnp.zeros_like(ss_ref)

    s_ref[...] += s_acc
    ss_ref[...] += ss_acc


def _passB(z1, s1, ss1, g11, bb11, w12, b12):
    vec = pl.BlockSpec((1, 16), lambda nb: (0, 0))
    return pl.pallas_call(
        _passB_body,
        grid=(NBE,),
        in_specs=[_eb_spec(), vec, vec, vec, vec,
                  pl.BlockSpec((16, 16), lambda nb: (0, 0)), vec],
        out_specs=[_eb_spec(), vec, vec],
        out_shape=[
            jax.ShapeDtypeStruct((EPAD, 16), jnp.float32),
            jax.ShapeDtypeStruct((1, 16), jnp.float32),
            jax.ShapeDtypeStruct((1, 16), jnp.float32),
        ],
    )(z1, s1, ss1, g11, bb11, w12, b12)


def _passC_body(z2_ref, s2_ref, ss2_ref, g12_ref, bb12_ref, w_ref, b_ref,
                x1_ref, x1t_ref):
    scale, shift = _bn_scale_shift(s2_ref[...], ss2_ref[...],
                                   g12_ref[...], bb12_ref[...])
    w = w_ref[...]
    b = b_ref[...]
    for q in range(RE // R):
        acc = None
        for k in range(K):
            s = q * K + k
            h = jnp.maximum(z2_ref[s * R:(s + 1) * R, :] * scale + shift,
                            0.0)
            m = _mm(h, w) + b
            acc = m if acc is None else jnp.maximum(acc, m)
        x1_ref[q * R:(q + 1) * R, :] = acc
        x1t_ref[:, q * R:(q + 1) * R] = acc.T


def _passC(z2, s2, ss2, g12, bb12, w13, b13):
    vec = pl.BlockSpec((1, 16), lambda nb: (0, 0))
    return pl.pallas_call(
        _passC_body,
        grid=(NBE,),
        in_specs=[_eb_spec(), vec, vec, vec, vec,
                  pl.BlockSpec((16, 16), lambda nb: (0, 0)), vec],
        out_specs=[pl.BlockSpec((RE, 16), lambda nb: (nb, 0)),
                   pl.BlockSpec((F, RE), lambda nb: (0, nb))],
        out_shape=[jax.ShapeDtypeStruct((NPAD, 16), jnp.float32),
                   jax.ShapeDtypeStruct((F, NPAD), jnp.float32)],
    )(z2, s2, ss2, g12, bb12, w13, b13)


def _passD_body(x1_ref, g_ref, w_ref, b_ref, x2_ref):
    w = w_ref[...]
    b = b_ref[...]
    for q in range(RE // R):
        x1b = x1_ref[q * R:(q + 1) * R, :]
        acc = None
        for k in range(K):
            s = q * K + k
            g = g_ref[s * R:(s + 1) * R, :]
            m = _mm(jnp.concatenate([x1b, g - x1b], axis=1), w) + b
            acc = m if acc is None else jnp.maximum(acc, m)
        x2_ref[q * R:(q + 1) * R, :] = acc


def _passD(x1, g2, w21, b21):
    return pl.pallas_call(
        _passD_body,
        grid=(NBE,),
        in_specs=[
            pl.BlockSpec((RE, 16), lambda nb: (nb, 0)),
            _eb_spec(),
            pl.BlockSpec((32, 32), lambda nb: (0, 0)),
            pl.BlockSpec((1, 32), lambda nb: (0, 0)),
        ],
        out_specs=pl.BlockSpec((RE, 32), lambda nb: (nb, 0)),
        out_shape=jax.ShapeDtypeStruct((NPAD, 32), jnp.float32),
    )(x1, g2, w21, b21)


def _pool_body(x1_ref, x2_ref, bcol_ref, w_ref, b_ref, pool_ref):
    nb = pl.program_id(0)
    o = _mm(jnp.concatenate([x1_ref[...], x2_ref[...]], axis=1),
            w_ref[...]) + b_ref[...]
    bcol = bcol_ref[...]
    parts = [
        jnp.max(jnp.where(bcol == s, o, -jnp.inf), axis=0, keepdims=True)
        for s in range(NSEG)
    ]
    pooled = jnp.concatenate(parts, axis=0)

    @pl.when(nb == 0)
    def _():
        pool_ref[...] = pooled

    @pl.when(nb > 0)
    def _():
        pool_ref[...] = jnp.maximum(pool_ref[...], pooled)


def _pool(x1, x2, bcol, wl, bl):
    return pl.pallas_call(
        _pool_body,
        grid=(NBE,),
        in_specs=[
            pl.BlockSpec((RE, 16), lambda nb: (nb, 0)),
            pl.BlockSpec((RE, 32), lambda nb: (nb, 0)),
            pl.BlockSpec((RE, 1), lambda nb: (nb, 0)),
            pl.BlockSpec((48, 128), lambda nb: (0, 0)),
            pl.BlockSpec((1, 128), lambda nb: (0, 0)),
        ],
        out_specs=pl.BlockSpec((NSEG, 128), lambda nb: (0, 0)),
        out_shape=jax.ShapeDtypeStruct((NSEG, 128), jnp.float32),
    )(x1, x2, bcol, wl, bl)


def _head_body(p_ref, w31_ref, b31_ref, w32_ref, b32_ref, w33_ref, b33_ref,
               out_ref):
    h = jnp.maximum(_mm(p_ref[...], w31_ref[...]) + b31_ref[...], 0.0)
    h = jnp.maximum(_mm(h, w32_ref[...]) + b32_ref[...], 0.0)
    z = _mm(h, w33_ref[...]) + b33_ref[...]
    mx = jnp.max(z, axis=1, keepdims=True)
    e = z - mx
    out_ref[...] = e - jnp.log(jnp.sum(jnp.exp(e), axis=1, keepdims=True))


def _head(pooled, w31, b31, w32, b32, w33, b33):
    return pl.pallas_call(
        _head_body,
        out_shape=jax.ShapeDtypeStruct((NSEG, 40), jnp.float32),
    )(pooled, w31, b31, w32, b32, w33, b33)


def _edge_order(idx):
    return idx.reshape(-1)


def kernel(pos, batch, w11, b11, g11, bb11, w12, b12, g12, bb12, w13, b13,
           w21, b21, wl, bl, w31, b31, w32, b32, w33, b33):
    batch = batch.astype(jnp.int32)

    pos16 = jnp.zeros((NPAD, 16), jnp.float32).at[:N, :3].set(pos)
    pos16T = jnp.zeros((16, NPAD), jnp.float32).at[:3, :N].set(pos.T)
    bcol = jnp.full((NPAD, 1), NSEG, jnp.int32).at[:N, 0].set(batch)

    ar = jnp.arange(NSEG)
    seg_start = jnp.sum(batch[None, :] < ar[:, None], axis=1).astype(jnp.int32)
    seg_end = jnp.sum(batch[None, :] <= ar[:, None], axis=1).astype(jnp.int32)
    rlo = jnp.zeros((NPAD,), jnp.int32).at[:N].set(
        seg_start[batch]).reshape(NB, 1, R)
    rhi = jnp.zeros((NPAD,), jnp.int32).at[:N].set(
        seg_end[batch]).reshape(NB, 1, R)
    blk0 = jnp.arange(NB, dtype=jnp.int32) * R
    bvec = bcol[:, 0]
    b_lo = bvec[blk0]
    b_hi = bvec[jnp.minimum(blk0 + R - 1, NPAD - 1)]
    lo_col = seg_start[jnp.minimum(b_lo, NSEG - 1)]
    hi_col = seg_end[jnp.minimum(b_hi, NSEG - 1)]
    lo_blk = lo_col // C
    nc = jnp.maximum((hi_col - lo_blk * C + C - 1) // C, 0)
    nc = jnp.where(b_lo >= NSEG, 0, nc).astype(jnp.int32)
    lo_blk = lo_blk.astype(jnp.int32)

    w11p = jnp.zeros((32, 16), jnp.float32)
    w11p = w11p.at[0:3].set(w11[0:3]).at[16:19].set(w11[3:6])

    r2 = lambda v: v.reshape(1, -1)

    idx1 = _knn(pos16T, pos16, rlo, rhi, lo_blk, nc)
    g1 = _gather_rows(pos16, _edge_order(idx1))
    z1, s1, ss1 = _passA(pos16, g1, w11p, r2(b11))
    z2, s2, ss2 = _passB(z1, s1, ss1, r2(g11), r2(bb11), w12, r2(b12))
    x1, x1T = _passC(z2, s2, ss2, r2(g12), r2(bb12), w13, r2(b13))

    idx2 = _knn(x1T, x1, rlo, rhi, lo_blk, nc)
    g2 = _gather_rows(x1, _edge_order(idx2))
    x2 = _passD(x1, g2, w21, r2(b21))

    pooled = _pool(x1, x2, bcol, wl, r2(bl))
    return _head(pooled, w31, r2(b31), w32, r2(b32), w33, r2(b33))
